# baseline (speedup 1.0000x reference)
import functools

import jax
import jax.numpy as jnp
import numpy as np
from jax import lax
from jax.experimental import pallas as pl
from jax.experimental.pallas import tpu as pltpu
from jax.experimental.pallas import tpu_sc as plsc

_TABLE_SIZE = 524288
_NUM_LEVELS = 16
_MIN_RES = 16
_MAX_RES = 2048
_FEATURE_DIM = 2
_MASK = _TABLE_SIZE - 1
_ROW = _TABLE_SIZE * _NUM_LEVELS
_P1 = int(np.uint32(2654435761).astype(np.int32))
_P2 = int(np.uint32(805459861).astype(np.int32))

_mesh = plsc.VectorSubcoreMesh(core_axis_name="c", subcore_axis_name="s")


@functools.partial(
    pl.kernel,
    mesh=_mesh,
    out_type=jax.ShapeDtypeStruct((32,), jnp.float32),
    scratch_types=[
        pltpu.VMEM((16,), jnp.float32),
        pltpu.VMEM((16,), jnp.float32),
        pltpu.VMEM((16, 16), jnp.float32),
        pltpu.VMEM((32,), jnp.float32),
        pltpu.SemaphoreType.DMA,
    ],
)
def _encode(x_hbm, scal_hbm, table_hbm, out_hbm, x_v, scal_v, rows_v,
            out_v, sem):
    first = (lax.axis_index("c") == 0) & (lax.axis_index("s") == 0)

    @pl.when(first)
    def _():
        pltpu.sync_copy(x_hbm, x_v)
        pltpu.sync_copy(scal_hbm, scal_v)

        lanes = lax.iota(jnp.int32, 16)
        scal = scal_v[...]
        xvec = x_v[...]

        def permute(vec, idx):
            dnums = lax.GatherDimensionNumbers(
                offset_dims=(), collapsed_slice_dims=(0,),
                start_index_map=(0,))
            return lax.gather(
                vec, idx[:, None], dnums, slice_sizes=(1,),
                mode=lax.GatherScatterMode.PROMISE_IN_BOUNDS)

        def bcast_x(d):
            return permute(xvec, lanes * 0 + d)

        fi = []
        ci = []
        for d in range(3):
            s = bcast_x(d) * scal
            f = s.astype(jnp.int32)
            c = jnp.where(s > f.astype(jnp.float32), f + 1, f)
            fi.append(f)
            ci.append(c)

        pc = [ci[0], ci[1] * _P1, ci[2] * _P2]
        pf = [fi[0], fi[1] * _P1, fi[2] * _P2]
        offs = lanes * _TABLE_SIZE

        def hfn(a, b, c):
            return ((a ^ b ^ c) & _MASK) + offs

        hs = [
            hfn(pc[0], pc[1], pc[2]),
            hfn(pc[0], pc[1], pf[2]),
            hfn(pc[0], pf[1], pc[2]),
            hfn(pf[0], pc[1], pc[2]),
            hfn(pc[0], pf[1], pf[2]),
            hfn(pf[0], pc[1], pf[2]),
            hfn(pf[0], pf[1], pc[2]),
            hfn(pf[0], pf[1], pf[2]),
        ]
        copies = []
        for k in range(8):
            copies.append(
                pltpu.async_copy(table_hbm.at[hs[k]], rows_v.at[k], sem))
            copies.append(
                pltpu.async_copy(table_hbm.at[hs[k] + _ROW], rows_v.at[k + 8],
                                 sem))
        for c in copies:
            c.wait()

        po = []
        for r in range(3):
            t = lanes + 16 * r
            three = lanes * 0 + 3
            xv = permute(xvec, lax.rem(t, three))
            sv = permute(scal, lax.div(t, three))
            s = xv * sv
            po.append(s - s.astype(jnp.int32).astype(jnp.float32))

        enc = []
        for f in range(2):
            g = [rows_v[f * 8 + k] for k in range(8)]
            f03 = g[0] * po[0] + g[3] * (1 - po[0])
            f12 = g[1] * po[0] + g[2] * (1 - po[0])
            f56 = g[5] * po[0] + g[6] * (1 - po[0])
            f47 = g[4] * po[0] + g[7] * (1 - po[0])
            f0312 = f03 * po[1] + f12 * (1 - po[1])
            f4756 = f47 * po[1] + f56 * (1 - po[1])
            enc.append(f0312 * po[2] + f4756 * (1 - po[2]))

        two = lanes * 0 + 2
        half = lax.div(lanes, two)
        even = lax.rem(lanes, two) == 0
        lo = jnp.where(even, permute(enc[0], half), permute(enc[1], half))
        hi = jnp.where(even, permute(enc[0], half + 8), permute(enc[1], half + 8))
        out_v[pl.ds(0, 16)] = lo
        out_v[pl.ds(16, 16)] = hi
        pltpu.sync_copy(out_v, out_hbm)


def kernel(x, hash_table):
    levels = jnp.arange(_NUM_LEVELS)
    growth = jnp.exp(
        (jnp.log(jnp.float32(_MAX_RES)) - jnp.log(jnp.float32(_MIN_RES)))
        / (_NUM_LEVELS - 1))
    scalings = jnp.floor(_MIN_RES * growth ** levels).astype(jnp.float32)
    x16 = jnp.zeros((16,), jnp.float32).at[:3].set(x)
    flat = hash_table.reshape(-1)
    return _encode(x16, scalings, flat)
